# baseline (speedup 1.0000x reference)
import functools

import numpy as np
import jax
import jax.numpy as jnp
from jax import lax
from jax.experimental import pallas as pl
from jax.experimental.pallas import tpu as pltpu
from jax.experimental.pallas import tpu_sc as plsc

_LMAX = 3
_CMAX = 128
_SRC = (_LMAX + 1) * _CMAX
_DST = (_LMAX + 1) ** 2 * _CMAX
_N = 50000

_NC, _NS = 2, 16
_NW = _NC * _NS
_R = 16
_NBLK = _N // _R
_BLK_PER_W = -(-_NBLK // _NW)
_IN_BLK = _R * _SRC
_OUT_BLK = _R * _DST

_IDX_NP = np.array(
    [l * _CMAX + c
     for l in range(_LMAX + 1)
     for c in range(_CMAX)
     for _ in range(2 * l + 1)],
    dtype=np.int32,
)
_TIDX_NP = (_IDX_NP // 128) * 1024 + _IDX_NP % 128


def _make_expand():
    mesh = plsc.VectorSubcoreMesh(
        core_axis_name="c", subcore_axis_name="s",
        num_cores=_NC, num_subcores=_NS)

    @functools.partial(
        pl.kernel,
        out_type=jax.ShapeDtypeStruct((_NBLK * _OUT_BLK,), jnp.float32),
        mesh=mesh,
        scratch_types=[
            pltpu.VMEM((_IN_BLK,), jnp.float32),
            pltpu.VMEM((_IN_BLK,), jnp.float32),
            pltpu.VMEM((_IN_BLK,), jnp.float32),
            pltpu.VMEM((_OUT_BLK,), jnp.float32),
            pltpu.VMEM((_OUT_BLK,), jnp.float32),
            pltpu.VMEM((_OUT_BLK,), jnp.float32),
            pltpu.SemaphoreType.DMA,
            pltpu.SemaphoreType.DMA,
            pltpu.SemaphoreType.DMA,
            pltpu.SemaphoreType.DMA,
            pltpu.SemaphoreType.DMA,
            pltpu.SemaphoreType.DMA,
        ],
        compiler_params=pltpu.CompilerParams(
            use_tc_tiling_on_sc=False, needs_layout_passes=False),
    )
    def expand(x_hbm, out_hbm,
               in_v0, in_v1, in_v2, out_v0, out_v1, out_v2,
               in_s0, in_s1, in_s2, out_s0, out_s1, out_s2):
        wid = lax.axis_index("s") * _NC + lax.axis_index("c")
        in_bufs, out_bufs = (in_v0, in_v1, in_v2), (out_v0, out_v1, out_v2)
        in_sems, out_sems = (in_s0, in_s1, in_s2), (out_s0, out_s1, out_s2)
        iroffs = [(r // 8) * 4096 + (r % 8) * 128 for r in range(_R)]
        soffs = [(r // 8) * 16384 + (r % 8) * 128 for r in range(_R)]
        lane = lax.iota(jnp.int32, 16)
        pats = {l: [(16 * t + lane) // (2 * l + 1)
                    for t in range(2 * l + 1)]
                for l in range(1, _LMAX + 1)}
        gb0 = [0, 8, 32, 72]

        for p in range(2):
            b = wid + p * _NW

            @pl.when(b < _NBLK)
            def _(b=b, p=p):
                pltpu.async_copy(
                    x_hbm.at[pl.ds(b * _IN_BLK, _IN_BLK)],
                    in_bufs[p], in_sems[p])

        def iter_body(i, carry):
            for q in range(3):
                k = i * 3 + q
                p = q
                b = wid + k * _NW

                @pl.when(b < _NBLK)
                def _(k=k, b=b, p=p):
                    pltpu.make_async_copy(
                        x_hbm.at[pl.ds(b * _IN_BLK, _IN_BLK)], in_bufs[p],
                        in_sems[p]).wait()

                    @pl.when(k >= 3)
                    def _():
                        pltpu.make_async_copy(
                            out_bufs[p],
                            out_hbm.at[pl.ds(b * _OUT_BLK, _OUT_BLK)],
                            out_sems[p]).wait()

                    b2 = wid + (k + 2) * _NW
                    p2 = (q + 2) % 3

                    @pl.when(b2 < _NBLK)
                    def _():
                        pltpu.async_copy(
                            x_hbm.at[pl.ds(b2 * _IN_BLK, _IN_BLK)],
                            in_bufs[p2], in_sems[p2])

                    for l in range(_LMAX + 1):
                        rep = 2 * l + 1

                        @plsc.parallel_loop(0, 8, 1, unroll=2)
                        def w_body(w, l=l, rep=rep, p=p):
                            wbase = l * 1024 + w * 16
                            wins = [in_bufs[p][pl.ds(wbase + iroffs[r], 16)]
                                    for r in range(_R)]
                            for t in range(rep):
                                ga = gb0[l] + w * rep + t
                                soff = ga * 16 + (ga >> 3) * 896
                                for r in range(_R):
                                    v = wins[r] if l == 0 else (
                                        wins[r].at[pats[l][t]].get(
                                            mode="promise_in_bounds"))
                                    out_bufs[p][
                                        pl.ds(soff + soffs[r], 16)] = v

                    pltpu.async_copy(
                        out_bufs[p],
                        out_hbm.at[pl.ds(b * _OUT_BLK, _OUT_BLK)],
                        out_sems[p])

            return carry

        lax.fori_loop(0, -(-_BLK_PER_W // 3), iter_body, 0)

        for k in range(_BLK_PER_W - 4, _BLK_PER_W):
            p = k % 3
            b = wid + k * _NW
            b3 = wid + (k + 3) * _NW

            @pl.when((b < _NBLK) & (b3 >= _NBLK))
            def _(b=b, p=p):
                pltpu.make_async_copy(
                    out_bufs[p], out_hbm.at[pl.ds(b * _OUT_BLK, _OUT_BLK)],
                    out_sems[p]).wait()

    return expand


_EXPAND = _make_expand()


def kernel(x):
    xt = x.reshape(_N // 8, 8, _SRC // 128, 128)
    xt = xt.transpose(0, 2, 1, 3).reshape(-1)
    outf = _EXPAND(xt)
    out = outf.reshape(_N // 8, _DST // 128, 8, 128)
    return out.transpose(0, 2, 1, 3).reshape(_N, _DST)


# speedup vs baseline: 1.0054x; 1.0054x over previous
import functools

import numpy as np
import jax
import jax.numpy as jnp
from jax import lax
from jax.experimental import pallas as pl
from jax.experimental.pallas import tpu as pltpu
from jax.experimental.pallas import tpu_sc as plsc

_LMAX = 3
_CMAX = 128
_SRC = (_LMAX + 1) * _CMAX
_DST = (_LMAX + 1) ** 2 * _CMAX
_N = 50000

_NC, _NS = 2, 16
_NW = _NC * _NS
_R = 24
_NBLK = 2080
_BLK_PER_W = _NBLK // _NW
_TAIL = (_N - _NBLK * _R) // 16
_IN_BLK = _R * _SRC
_OUT_BLK = _R * _DST

_IDX_NP = np.array(
    [l * _CMAX + c
     for l in range(_LMAX + 1)
     for c in range(_CMAX)
     for _ in range(2 * l + 1)],
    dtype=np.int32,
)
_TIDX_NP = (_IDX_NP // 128) * 1024 + _IDX_NP % 128


def _make_expand():
    mesh = plsc.VectorSubcoreMesh(
        core_axis_name="c", subcore_axis_name="s",
        num_cores=_NC, num_subcores=_NS)

    @functools.partial(
        pl.kernel,
        out_type=jax.ShapeDtypeStruct((_N * _DST,), jnp.float32),
        mesh=mesh,
        scratch_types=[
            pltpu.VMEM((_DST,), jnp.int32),
            pltpu.VMEM((_IN_BLK,), jnp.float32),
            pltpu.VMEM((_IN_BLK,), jnp.float32),
            pltpu.VMEM((_OUT_BLK,), jnp.float32),
            pltpu.VMEM((_OUT_BLK,), jnp.float32),
            pltpu.SemaphoreType.DMA,
            pltpu.SemaphoreType.DMA,
            pltpu.SemaphoreType.DMA,
            pltpu.SemaphoreType.DMA,
        ],
        compiler_params=pltpu.CompilerParams(
            use_tc_tiling_on_sc=False, needs_layout_passes=False),
    )
    def expand(x_hbm, idx_hbm, out_hbm,
               idx_v, in_v0, in_v1, out_v0, out_v1,
               in_s0, in_s1, out_s0, out_s1):
        wid = lax.axis_index("s") * _NC + lax.axis_index("c")
        in_bufs, out_bufs = (in_v0, in_v1), (out_v0, out_v1)
        in_sems, out_sems = (in_s0, in_s1), (out_s0, out_s1)
        pltpu.sync_copy(idx_hbm, idx_v)
        roffs = [jnp.full((16,), (r // 8) * 4096 + (r % 8) * 128, jnp.int32)
                 for r in range(_R)]
        soffs = [(r // 8) * 16384 + (r % 8) * 128 for r in range(_R)]

        def expand_block(p, nrows):
            @plsc.parallel_loop(0, _DST // 16, 1, unroll=2)
            def g_body(g):
                base = g * 16
                soff = base + (g >> 3) * 896
                tg = idx_v[pl.ds(base, 16)]
                for r in range(nrows):
                    out_bufs[p][pl.ds(soff + soffs[r], 16)] = (
                        plsc.load_gather(in_bufs[p], [tg + roffs[r]]))

        for p in range(2):
            b = wid + p * _NW
            pltpu.async_copy(
                x_hbm.at[pl.ds(b * _IN_BLK, _IN_BLK)], in_bufs[p], in_sems[p])

        def iter_body(i, carry):
            for p in range(2):
                k = i * 2 + p
                b = wid + k * _NW
                pltpu.make_async_copy(
                    x_hbm.at[pl.ds(b * _IN_BLK, _IN_BLK)], in_bufs[p],
                    in_sems[p]).wait()

                @pl.when(k >= 2)
                def _(k=k, b=b, p=p):
                    pltpu.make_async_copy(
                        out_bufs[p], out_hbm.at[pl.ds(b * _OUT_BLK, _OUT_BLK)],
                        out_sems[p]).wait()

                expand_block(p, _R)
                pltpu.async_copy(
                    out_bufs[p], out_hbm.at[pl.ds(b * _OUT_BLK, _OUT_BLK)],
                    out_sems[p])
                b2 = wid + (k + 2) * _NW

                @pl.when(b2 < _NBLK)
                def _(b2=b2, p=p):
                    pltpu.async_copy(
                        x_hbm.at[pl.ds(b2 * _IN_BLK, _IN_BLK)],
                        in_bufs[p], in_sems[p])

            return carry

        lax.fori_loop(0, _BLK_PER_W // 2, iter_body, 0)

        k_last = _BLK_PER_W - 1
        p_last = k_last % 2
        b_last = wid + k_last * _NW
        pltpu.make_async_copy(
            x_hbm.at[pl.ds(b_last * _IN_BLK, _IN_BLK)], in_bufs[p_last],
            in_sems[p_last]).wait()
        pltpu.make_async_copy(
            out_bufs[p_last],
            out_hbm.at[pl.ds((b_last - 2 * _NW) * _OUT_BLK, _OUT_BLK)],
            out_sems[p_last]).wait()
        expand_block(p_last, _R)
        pltpu.async_copy(
            out_bufs[p_last],
            out_hbm.at[pl.ds(b_last * _OUT_BLK, _OUT_BLK)], out_sems[p_last])
        for k in (k_last - 1, k_last):
            p = k % 2
            b = wid + k * _NW
            pltpu.make_async_copy(
                out_bufs[p], out_hbm.at[pl.ds(b * _OUT_BLK, _OUT_BLK)],
                out_sems[p]).wait()

        @pl.when(wid < _TAIL)
        def _():
            tin = _NBLK * _IN_BLK + wid * 16 * _SRC
            tout = _NBLK * _OUT_BLK + wid * 16 * _DST
            pltpu.async_copy(
                x_hbm.at[pl.ds(tin, 16 * _SRC)],
                in_bufs[0].at[pl.ds(0, 16 * _SRC)], in_sems[0])
            pltpu.make_async_copy(
                x_hbm.at[pl.ds(tin, 16 * _SRC)],
                in_bufs[0].at[pl.ds(0, 16 * _SRC)], in_sems[0]).wait()
            expand_block(0, 16)
            pltpu.async_copy(
                out_bufs[0].at[pl.ds(0, 16 * _DST)],
                out_hbm.at[pl.ds(tout, 16 * _DST)], out_sems[0])
            pltpu.make_async_copy(
                out_bufs[0].at[pl.ds(0, 16 * _DST)],
                out_hbm.at[pl.ds(tout, 16 * _DST)], out_sems[0]).wait()

    return expand


_EXPAND = _make_expand()


def kernel(x):
    xt = x.reshape(_N // 8, 8, _SRC // 128, 128)
    xt = xt.transpose(0, 2, 1, 3).reshape(-1)
    outf = _EXPAND(xt, jnp.asarray(_TIDX_NP))
    out = outf.reshape(_N // 8, _DST // 128, 8, 128)
    return out.transpose(0, 2, 1, 3).reshape(_N, _DST)


# speedup vs baseline: 1.0234x; 1.0179x over previous
import functools

import numpy as np
import jax
import jax.numpy as jnp
from jax import lax
from jax.experimental import pallas as pl
from jax.experimental.pallas import tpu as pltpu
from jax.experimental.pallas import tpu_sc as plsc

_LMAX = 3
_CMAX = 128
_SRC = (_LMAX + 1) * _CMAX
_DST = (_LMAX + 1) ** 2 * _CMAX
_N = 50000

_NC, _NS = 2, 16
_NW = _NC * _NS
_R = 16
_NBLK = _N // _R
_BLK_PER_W = -(-_NBLK // _NW)
_IN_BLK = _R * _SRC
_OUT_BLK = _R * _DST

_IDX_NP = np.array(
    [l * _CMAX + c
     for l in range(_LMAX + 1)
     for c in range(_CMAX)
     for _ in range(2 * l + 1)],
    dtype=np.int32,
)
_TIDX_NP = (_IDX_NP // 128) * 1024 + _IDX_NP % 128


def _make_expand():
    mesh = plsc.VectorSubcoreMesh(
        core_axis_name="c", subcore_axis_name="s",
        num_cores=_NC, num_subcores=_NS)

    @functools.partial(
        pl.kernel,
        out_type=jax.ShapeDtypeStruct((_NBLK * _OUT_BLK,), jnp.float32),
        mesh=mesh,
        scratch_types=[
            pltpu.VMEM((_DST,), jnp.int32),
            pltpu.VMEM((_IN_BLK,), jnp.float32),
            pltpu.VMEM((_IN_BLK,), jnp.float32),
            pltpu.VMEM((_IN_BLK,), jnp.float32),
            pltpu.VMEM((_OUT_BLK,), jnp.float32),
            pltpu.VMEM((_OUT_BLK,), jnp.float32),
            pltpu.VMEM((_OUT_BLK,), jnp.float32),
            pltpu.SemaphoreType.DMA,
            pltpu.SemaphoreType.DMA,
            pltpu.SemaphoreType.DMA,
            pltpu.SemaphoreType.DMA,
            pltpu.SemaphoreType.DMA,
            pltpu.SemaphoreType.DMA,
        ],
        compiler_params=pltpu.CompilerParams(
            use_tc_tiling_on_sc=False, needs_layout_passes=False),
    )
    def expand(x_hbm, idx_hbm, out_hbm,
               idx_v, in_v0, in_v1, in_v2, out_v0, out_v1, out_v2,
               in_s0, in_s1, in_s2, out_s0, out_s1, out_s2):
        wid = lax.axis_index("s") * _NC + lax.axis_index("c")
        in_bufs, out_bufs = (in_v0, in_v1, in_v2), (out_v0, out_v1, out_v2)
        in_sems, out_sems = (in_s0, in_s1, in_s2), (out_s0, out_s1, out_s2)
        pltpu.sync_copy(idx_hbm, idx_v)
        roffs = [jnp.full((16,), (r // 8) * 4096 + (r % 8) * 128, jnp.int32)
                 for r in range(_R)]
        soffs = [(r // 8) * 16384 + (r % 8) * 128 for r in range(_R)]

        for p in range(2):
            b = wid + p * _NW

            @pl.when(b < _NBLK)
            def _(b=b, p=p):
                pltpu.async_copy(
                    x_hbm.at[pl.ds(b * _IN_BLK, _IN_BLK)],
                    in_bufs[p], in_sems[p])

        def iter_body(i, carry):
            for q in range(3):
                k = i * 3 + q
                p = q
                b = wid + k * _NW

                @pl.when(b < _NBLK)
                def _(k=k, b=b, p=p):
                    pltpu.make_async_copy(
                        x_hbm.at[pl.ds(b * _IN_BLK, _IN_BLK)], in_bufs[p],
                        in_sems[p]).wait()

                    @pl.when(k >= 3)
                    def _():
                        pltpu.make_async_copy(
                            out_bufs[p],
                            out_hbm.at[pl.ds(b * _OUT_BLK, _OUT_BLK)],
                            out_sems[p]).wait()

                    b2 = wid + (k + 2) * _NW
                    p2 = (q + 2) % 3

                    @pl.when(b2 < _NBLK)
                    def _():
                        pltpu.async_copy(
                            x_hbm.at[pl.ds(b2 * _IN_BLK, _IN_BLK)],
                            in_bufs[p2], in_sems[p2])

                    @plsc.parallel_loop(0, _DST // 16, 1, unroll=2)
                    def g_body(g):
                        base = g * 16
                        soff = base + (g >> 3) * 896
                        tg = idx_v[pl.ds(base, 16)]
                        for r in range(_R):
                            out_bufs[p][pl.ds(soff + soffs[r], 16)] = (
                                plsc.load_gather(in_bufs[p], [tg + roffs[r]]))

                    pltpu.async_copy(
                        out_bufs[p],
                        out_hbm.at[pl.ds(b * _OUT_BLK, _OUT_BLK)],
                        out_sems[p])

            return carry

        lax.fori_loop(0, -(-_BLK_PER_W // 3), iter_body, 0)

        for k in range(_BLK_PER_W - 4, _BLK_PER_W):
            p = k % 3
            b = wid + k * _NW
            b3 = wid + (k + 3) * _NW

            @pl.when((b < _NBLK) & (b3 >= _NBLK))
            def _(b=b, p=p):
                pltpu.make_async_copy(
                    out_bufs[p], out_hbm.at[pl.ds(b * _OUT_BLK, _OUT_BLK)],
                    out_sems[p]).wait()

    return expand


_EXPAND = _make_expand()


def kernel(x):
    xt = x.reshape(_N // 8, 8, _SRC // 128, 128)
    xt = xt.transpose(0, 2, 1, 3).reshape(-1)
    outf = _EXPAND(xt, jnp.asarray(_TIDX_NP))
    out = outf.reshape(_N // 8, _DST // 128, 8, 128)
    return out.transpose(0, 2, 1, 3).reshape(_N, _DST)
